# baseline (speedup 1.0000x reference)
import functools

import jax
import jax.numpy as jnp
from jax import lax
from jax.experimental import pallas as pl
from jax.experimental.pallas import tpu as pltpu
from jax.experimental.pallas import tpu_sc as plsc

_B, _N, _M = 4, 8192, 2048
_CS = 512
_CO = 1024
_CI = 2 * _CS
_CSH = _CS // 2
_R = _B * _N
_NBQ = 1024
_NBLK = _N // _NBQ
_NMM = 1024
_MMB = _N // _NMM


def _argmin_body(q_ref, st_ref, idx_ref, *, boff):
    q = q_ref[0]
    st = st_ref[0]
    st2 = -2.0 * st
    mat = jnp.dot(q, st2, preferred_element_type=jnp.float32)
    qn = jnp.sum(q * q, axis=1, keepdims=True)
    sn = 0.25 * jnp.sum(st2 * st2, axis=0, keepdims=True)
    sq = mat + qn
    sq = sq + sn
    idx = jnp.argmin(sq, axis=1).astype(jnp.int32)[:, None]
    idx_ref[0] = idx + boff


@functools.cache
def _make_argmin(b):
    return pl.pallas_call(
        functools.partial(_argmin_body, boff=b * _M),
        grid=(_NBLK,),
        in_specs=[
            pl.BlockSpec((1, _NBQ, 8), lambda i: (i, 0, 0)),
            pl.BlockSpec((1, 8, _M), lambda i: (0, 0, 0)),
        ],
        out_specs=pl.BlockSpec((1, _NBQ, 1), lambda i: (i, 0, 0)),
        out_shape=jax.ShapeDtypeStruct((_NBLK, _NBQ, 1), jnp.int32),
    )


def _pack_body(x_ref, o_ref):
    xl = x_ref[:, :_CSH].astype(jnp.bfloat16)
    xh = x_ref[:, _CSH:].astype(jnp.bfloat16)
    bl = lax.bitcast_convert_type(xl, jnp.int16).astype(jnp.int32)
    bh = lax.bitcast_convert_type(xh, jnp.int16).astype(jnp.int32)
    o_ref[...] = (bh << 16) | (bl & 0xFFFF)


_pack_call = pl.pallas_call(
    _pack_body,
    grid=(4,),
    in_specs=[pl.BlockSpec((_B * _M // 4, _CS), lambda i: (i, 0))],
    out_specs=pl.BlockSpec((_B * _M // 4, _CSH), lambda i: (i, 0)),
    out_shape=jax.ShapeDtypeStruct((_B * _M, _CSH), jnp.int32),
)


_NC, _NS = 2, 16
_NW = _NC * _NS
_CH = 128


@functools.cache
def _make_sc_gather(rows):
    rpw = rows // _NW
    nch = rpw // _CH
    mesh = plsc.VectorSubcoreMesh(core_axis_name="c", subcore_axis_name="s")

    @functools.partial(
        pl.kernel,
        mesh=mesh,
        out_type=jax.ShapeDtypeStruct((rows, _CSH), jnp.int32),
        scratch_types=[
            pltpu.VMEM((rpw,), jnp.int32),
            pltpu.VMEM((_CH, _CSH), jnp.int32),
            pltpu.VMEM((_CH, _CSH), jnp.int32),
            pltpu.VMEM((_CH, _CSH), jnp.int32),
            pltpu.SemaphoreType.DMA,
            pltpu.SemaphoreType.DMA,
            pltpu.SemaphoreType.DMA,
            pltpu.SemaphoreType.DMA,
            pltpu.SemaphoreType.DMA,
            pltpu.SemaphoreType.DMA,
        ],
    )
    def _sc_gather(idx_hbm, table_hbm, out_hbm, idx_v, r0, r1, r2, g0, g1, g2, o0, o1, o2):
        wid = lax.axis_index("s") * _NC + lax.axis_index("c")
        base = wid * rpw
        bufs = (r0, r1, r2)
        gsems = (g0, g1, g2)
        osems = (o0, o1, o2)
        pltpu.sync_copy(idx_hbm.at[pl.ds(base, rpw)], idx_v)

        def gather_copy(c):
            b = c % 3
            return pltpu.make_async_copy(
                table_hbm.at[idx_v.at[pl.ds(c * _CH, _CH)]], bufs[b], gsems[b]
            )

        def out_copy(c):
            b = c % 3
            return pltpu.make_async_copy(
                bufs[b], out_hbm.at[pl.ds(base + c * _CH, _CH)], osems[b]
            )

        for c in range(min(3, nch)):
            gather_copy(c).start()
        for c in range(nch):
            gather_copy(c).wait()
            out_copy(c).start()
            if c + 3 < nch:
                out_copy(c).wait()
                gather_copy(c + 3).start()
        for c in range(max(nch - 3, 0), nch):
            out_copy(c).wait()

    return _sc_gather


def _mm_body(g_ref, s_ref, wt_ref, b_ref, o_ref, st_ref):
    i = pl.program_id(0)
    p = g_ref[...]
    glo = lax.bitcast_convert_type(p << 16, jnp.float32).astype(jnp.bfloat16)
    ghi = lax.bitcast_convert_type(p & jnp.int32(-65536), jnp.float32).astype(jnp.bfloat16)
    s = s_ref[...].astype(jnp.bfloat16)
    out = jnp.dot(glo, wt_ref[:_CSH, :], preferred_element_type=jnp.float32)
    out = out + jnp.dot(ghi, wt_ref[_CSH:_CS, :], preferred_element_type=jnp.float32)
    out = out + jnp.dot(s, wt_ref[_CS:, :], preferred_element_type=jnp.float32)
    out = out + b_ref[0:1, :]
    o_ref[...] = out.astype(jnp.bfloat16)
    ssum = jnp.sum(out, axis=0, keepdims=True)
    ssq = jnp.sum(out * out, axis=0, keepdims=True)
    acc = jnp.concatenate([ssum, ssq], axis=0)

    @pl.when(i == 0)
    def _():
        st_ref[...] = jnp.zeros_like(st_ref)

    st_ref[0:2, :] = st_ref[0:2, :] + acc


_mm_call = pl.pallas_call(
    _mm_body,
    grid=(_MMB,),
    in_specs=[
        pl.BlockSpec((_NMM, _CSH), lambda i: (i, 0)),
        pl.BlockSpec((_NMM, _CS), lambda i: (i, 0)),
        pl.BlockSpec((_CI, _CO), lambda i: (0, 0)),
        pl.BlockSpec((1, _CO), lambda i: (0, 0)),
    ],
    out_specs=[
        pl.BlockSpec((_NMM, _CO), lambda i: (i, 0)),
        pl.BlockSpec((8, _CO), lambda i: (0, 0)),
    ],
    out_shape=[
        jax.ShapeDtypeStruct((_N, _CO), jnp.bfloat16),
        jax.ShapeDtypeStruct((8, _CO), jnp.float32),
    ],
)


def _norm_body(o0_ref, o1_ref, o2_ref, o3_ref, st0_ref, st1_ref, st2_ref,
               st3_ref, g_ref, b_ref, y_ref):
    i = pl.program_id(0)
    bidx = i // _MMB
    st = st0_ref[0:2, :] + st1_ref[0:2, :] + st2_ref[0:2, :] + st3_ref[0:2, :]
    inv_r = 1.0 / _R
    mean = st[0:1, :] * inv_r
    var = st[1:2, :] * inv_r - mean * mean
    scale = g_ref[0:1, :] / jnp.sqrt(var + 1e-5)
    o = o0_ref[...]
    o = jnp.where(bidx == 1, o1_ref[...], o)
    o = jnp.where(bidx == 2, o2_ref[...], o)
    o = jnp.where(bidx == 3, o3_ref[...], o)
    y = (o.astype(jnp.float32) - mean) * scale + b_ref[0:1, :]
    y_ref[...] = jnp.where(y >= 0, y, 0.1 * y)


def _clamped(b):
    def index_map(i):
        return (jnp.clip(i - b * _MMB, 0, _MMB - 1), 0)
    return index_map


_norm_call = pl.pallas_call(
    _norm_body,
    grid=(_B * _MMB,),
    in_specs=[
        pl.BlockSpec((_NMM, _CO), _clamped(0)),
        pl.BlockSpec((_NMM, _CO), _clamped(1)),
        pl.BlockSpec((_NMM, _CO), _clamped(2)),
        pl.BlockSpec((_NMM, _CO), _clamped(3)),
        pl.BlockSpec((8, _CO), lambda i: (0, 0)),
        pl.BlockSpec((8, _CO), lambda i: (0, 0)),
        pl.BlockSpec((8, _CO), lambda i: (0, 0)),
        pl.BlockSpec((8, _CO), lambda i: (0, 0)),
        pl.BlockSpec((1, _CO), lambda i: (0, 0)),
        pl.BlockSpec((1, _CO), lambda i: (0, 0)),
    ],
    out_specs=pl.BlockSpec((_NMM, _CO), lambda i: (i, 0)),
    out_shape=jax.ShapeDtypeStruct((_R, _CO), jnp.float32),
)


def kernel(query_points, support_points, support_features, skip_features, W, bias, gamma, beta):
    qp = jnp.pad(query_points, ((0, 0), (0, 0), (0, 5)))
    qp = qp.reshape(_B, _NBLK, _NBQ, 8)
    spt = jnp.pad(jnp.transpose(support_points, (0, 2, 1)), ((0, 0), (0, 5), (0, 0)))
    table = _pack_call(support_features.reshape(_B * _M, _CS))
    skip = skip_features.reshape(_B, _N, _CS)
    wt = W.T.astype(jnp.bfloat16)
    bias2 = bias.reshape(1, _CO)
    gather = _make_sc_gather(_N)
    outs, stats = [], []
    for b in range(_B):
        idx_b = _make_argmin(b)(qp[b], spt[b:b + 1]).reshape(_N)
        g_b = gather(idx_b, table)
        o_b, st_b = _mm_call(g_b, skip[b], wt, bias2)
        outs.append(o_b)
        stats.append(st_b)
    y = _norm_call(*outs, *stats, gamma.reshape(1, _CO), beta.reshape(1, _CO))
    return y.reshape(_B, _N, _CO)


# speedup vs baseline: 1.2255x; 1.2255x over previous
import functools

import jax
import jax.numpy as jnp
from jax import lax
from jax.experimental import pallas as pl
from jax.experimental.pallas import tpu as pltpu
from jax.experimental.pallas import tpu_sc as plsc

_B, _N, _M = 4, 8192, 2048
_CS = 512
_CO = 1024
_CI = 2 * _CS
_CSH = _CS // 2
_R = _B * _N
_NBQ = 2048
_NBLK = _N // _NBQ
_NMM = 1024


def _argmin_body(q_ref, st_ref, idx_ref):
    b = pl.program_id(0) // _NBLK
    q = q_ref[0]
    st = st_ref[0]
    st2 = -2.0 * st
    mat = jnp.dot(q, st2, preferred_element_type=jnp.float32)
    qn = jnp.sum(q * q, axis=1, keepdims=True)
    sn = 0.25 * jnp.sum(st2 * st2, axis=0, keepdims=True)
    sq = mat + qn
    sq = sq + sn
    idx = jnp.argmin(sq, axis=1).astype(jnp.int32)[:, None]
    idx_ref[0] = idx + b * _M


_argmin_call = pl.pallas_call(
    _argmin_body,
    grid=(_B * _NBLK,),
    in_specs=[
        pl.BlockSpec((1, _NBQ, 8), lambda i: (i, 0, 0)),
        pl.BlockSpec((1, 8, _M), lambda i: (i // _NBLK, 0, 0)),
    ],
    out_specs=pl.BlockSpec((1, _NBQ, 1), lambda i: (i, 0, 0)),
    out_shape=jax.ShapeDtypeStruct((_B * _NBLK, _NBQ, 1), jnp.int32),
)


def _pack_body(x_ref, o_ref):
    xl = x_ref[:, :_CSH].astype(jnp.bfloat16)
    xh = x_ref[:, _CSH:].astype(jnp.bfloat16)
    bl = lax.bitcast_convert_type(xl, jnp.int16).astype(jnp.int32)
    bh = lax.bitcast_convert_type(xh, jnp.int16).astype(jnp.int32)
    o_ref[...] = (bh << 16) | (bl & 0xFFFF)


_pack_call = pl.pallas_call(
    _pack_body,
    grid=(4,),
    in_specs=[pl.BlockSpec((_B * _M // 4, _CS), lambda i: (i, 0))],
    out_specs=pl.BlockSpec((_B * _M // 4, _CSH), lambda i: (i, 0)),
    out_shape=jax.ShapeDtypeStruct((_B * _M, _CSH), jnp.int32),
)


_NC, _NS = 2, 16
_NW = _NC * _NS
_RPW = _R // _NW
_CH = 128
_NCH = _RPW // _CH

@functools.cache
def _make_sc_gather():
    mesh = plsc.VectorSubcoreMesh(core_axis_name="c", subcore_axis_name="s")

    @functools.partial(
        pl.kernel,
        mesh=mesh,
        out_type=jax.ShapeDtypeStruct((_R, _CSH), jnp.int32),
        scratch_types=[
            pltpu.VMEM((_RPW,), jnp.int32),
            pltpu.VMEM((_CH, _CSH), jnp.int32),
            pltpu.VMEM((_CH, _CSH), jnp.int32),
            pltpu.VMEM((_CH, _CSH), jnp.int32),
            pltpu.SemaphoreType.DMA,
            pltpu.SemaphoreType.DMA,
            pltpu.SemaphoreType.DMA,
            pltpu.SemaphoreType.DMA,
            pltpu.SemaphoreType.DMA,
            pltpu.SemaphoreType.DMA,
        ],
    )
    def _sc_gather(idx_hbm, table_hbm, out_hbm, idx_v, r0, r1, r2, g0, g1, g2, o0, o1, o2):
        wid = lax.axis_index("s") * _NC + lax.axis_index("c")
        base = wid * _RPW
        bufs = (r0, r1, r2)
        gsems = (g0, g1, g2)
        osems = (o0, o1, o2)
        pltpu.sync_copy(idx_hbm.at[pl.ds(base, _RPW)], idx_v)

        def gather_copy(c):
            b = c % 3
            return pltpu.make_async_copy(
                table_hbm.at[idx_v.at[pl.ds(c * _CH, _CH)]], bufs[b], gsems[b]
            )

        def out_copy(c):
            b = c % 3
            return pltpu.make_async_copy(
                bufs[b], out_hbm.at[pl.ds(base + c * _CH, _CH)], osems[b]
            )

        for c in range(min(3, _NCH)):
            gather_copy(c).start()
        for c in range(_NCH):
            gather_copy(c).wait()
            out_copy(c).start()
            if c + 3 < _NCH:
                out_copy(c).wait()
                gather_copy(c + 3).start()
        for c in range(max(_NCH - 3, 0), _NCH):
            out_copy(c).wait()

    return _sc_gather


def _mm_body(g_ref, s_ref, wt_ref, o_ref, st_ref):
    i = pl.program_id(0)
    p = g_ref[...]
    glo = lax.bitcast_convert_type(p << 16, jnp.float32).astype(jnp.bfloat16)
    ghi = lax.bitcast_convert_type(p & jnp.int32(-65536), jnp.float32).astype(jnp.bfloat16)
    s = s_ref[...].astype(jnp.bfloat16)
    out = jnp.dot(glo, wt_ref[:_CSH, :], preferred_element_type=jnp.float32)
    out = out + jnp.dot(ghi, wt_ref[_CSH:_CS, :], preferred_element_type=jnp.float32)
    out = out + jnp.dot(s, wt_ref[_CS:, :], preferred_element_type=jnp.float32)
    o_ref[...] = out.astype(jnp.bfloat16)
    ssum = jnp.sum(out, axis=0, keepdims=True)
    ssq = jnp.sum(out * out, axis=0, keepdims=True)
    acc = jnp.concatenate([ssum, ssq], axis=0)

    @pl.when(i == 0)
    def _():
        st_ref[...] = jnp.zeros_like(st_ref)

    st_ref[0:2, :] = st_ref[0:2, :] + acc


_mm_call = pl.pallas_call(
    _mm_body,
    grid=(_R // _NMM,),
    in_specs=[
        pl.BlockSpec((_NMM, _CSH), lambda i: (i, 0)),
        pl.BlockSpec((_NMM, _CS), lambda i: (i, 0)),
        pl.BlockSpec((_CI, _CO), lambda i: (0, 0)),
    ],
    out_specs=[
        pl.BlockSpec((_NMM, _CO), lambda i: (i, 0)),
        pl.BlockSpec((8, _CO), lambda i: (0, 0)),
    ],
    out_shape=[
        jax.ShapeDtypeStruct((_R, _CO), jnp.bfloat16),
        jax.ShapeDtypeStruct((8, _CO), jnp.float32),
    ],
)


def _norm_body(o_ref, st_ref, g_ref, b_ref, y_ref):
    inv_r = 1.0 / _R
    mean = st_ref[0:1, :] * inv_r
    var = st_ref[1:2, :] * inv_r - mean * mean
    scale = g_ref[0:1, :] / jnp.sqrt(var + 1e-5)
    y = (o_ref[...].astype(jnp.float32) - mean) * scale + b_ref[0:1, :]
    y_ref[...] = jnp.where(y >= 0, y, 0.1 * y)


_norm_call = pl.pallas_call(
    _norm_body,
    grid=(_R // _NMM,),
    in_specs=[
        pl.BlockSpec((_NMM, _CO), lambda i: (i, 0)),
        pl.BlockSpec((8, _CO), lambda i: (0, 0)),
        pl.BlockSpec((1, _CO), lambda i: (0, 0)),
        pl.BlockSpec((1, _CO), lambda i: (0, 0)),
    ],
    out_specs=pl.BlockSpec((_NMM, _CO), lambda i: (i, 0)),
    out_shape=jax.ShapeDtypeStruct((_R, _CO), jnp.float32),
)


def kernel(query_points, support_points, support_features, skip_features, W, bias, gamma, beta):
    qp = jnp.pad(query_points, ((0, 0), (0, 0), (0, 5)))
    qp = qp.reshape(_B * _NBLK, _NBQ, 8)
    spt = jnp.pad(jnp.transpose(support_points, (0, 2, 1)), ((0, 0), (0, 5), (0, 0)))
    idx = _argmin_call(qp, spt).reshape(_R)
    table = _pack_call(support_features.reshape(_B * _M, _CS))
    g = _make_sc_gather()(idx, table)
    skip = skip_features.reshape(_R, _CS)
    out_raw, st = _mm_call(g, skip, W.T.astype(jnp.bfloat16))
    y = _norm_call(out_raw, st, gamma.reshape(1, _CO), beta.reshape(1, _CO))
    return y.reshape(_B, _N, _CO)


# speedup vs baseline: 1.2608x; 1.0288x over previous
import functools

import jax
import jax.numpy as jnp
from jax import lax
from jax.experimental import pallas as pl
from jax.experimental.pallas import tpu as pltpu
from jax.experimental.pallas import tpu_sc as plsc

_B, _N, _M = 4, 8192, 2048
_CS = 512
_CO = 1024
_CI = 2 * _CS
_CSH = _CS // 2
_R = _B * _N
_NBQ = 2048
_NBLK = _N // _NBQ
_NMM = 2048


def _argmin_body(q_ref, st_ref, idx_ref):
    b = pl.program_id(0) // _NBLK
    q = q_ref[0]
    st = st_ref[0]
    st2 = -2.0 * st
    mat = jnp.dot(q, st2, preferred_element_type=jnp.float32)
    qn = jnp.sum(q * q, axis=1, keepdims=True)
    sn = 0.25 * jnp.sum(st2 * st2, axis=0, keepdims=True)
    sq = mat + qn
    sq = sq + sn
    idx = jnp.argmin(sq, axis=1).astype(jnp.int32)[:, None]
    idx_ref[0] = idx + b * _M


_argmin_call = pl.pallas_call(
    _argmin_body,
    grid=(_B * _NBLK,),
    in_specs=[
        pl.BlockSpec((1, _NBQ, 8), lambda i: (i, 0, 0)),
        pl.BlockSpec((1, 8, _M), lambda i: (i // _NBLK, 0, 0)),
    ],
    out_specs=pl.BlockSpec((1, _NBQ, 1), lambda i: (i, 0, 0)),
    out_shape=jax.ShapeDtypeStruct((_B * _NBLK, _NBQ, 1), jnp.int32),
)


def _pack_body(x_ref, o_ref):
    xl = x_ref[:, :_CSH].astype(jnp.bfloat16)
    xh = x_ref[:, _CSH:].astype(jnp.bfloat16)
    bl = lax.bitcast_convert_type(xl, jnp.int16).astype(jnp.int32)
    bh = lax.bitcast_convert_type(xh, jnp.int16).astype(jnp.int32)
    o_ref[...] = (bh << 16) | (bl & 0xFFFF)


_pack_call = pl.pallas_call(
    _pack_body,
    grid=(4,),
    in_specs=[pl.BlockSpec((_B * _M // 4, _CS), lambda i: (i, 0))],
    out_specs=pl.BlockSpec((_B * _M // 4, _CSH), lambda i: (i, 0)),
    out_shape=jax.ShapeDtypeStruct((_B * _M, _CSH), jnp.int32),
)


_NC, _NS = 2, 16
_NW = _NC * _NS
_RPW = _R // _NW
_CH = 128
_NCH = _RPW // _CH

@functools.cache
def _make_sc_gather():
    mesh = plsc.VectorSubcoreMesh(core_axis_name="c", subcore_axis_name="s")

    @functools.partial(
        pl.kernel,
        mesh=mesh,
        out_type=jax.ShapeDtypeStruct((_R, _CSH), jnp.int32),
        scratch_types=[
            pltpu.VMEM((_RPW,), jnp.int32),
            pltpu.VMEM((_CH, _CSH), jnp.int32),
            pltpu.VMEM((_CH, _CSH), jnp.int32),
            pltpu.VMEM((_CH, _CSH), jnp.int32),
            pltpu.SemaphoreType.DMA,
            pltpu.SemaphoreType.DMA,
            pltpu.SemaphoreType.DMA,
            pltpu.SemaphoreType.DMA,
            pltpu.SemaphoreType.DMA,
            pltpu.SemaphoreType.DMA,
        ],
    )
    def _sc_gather(idx_hbm, table_hbm, out_hbm, idx_v, r0, r1, r2, g0, g1, g2, o0, o1, o2):
        wid = lax.axis_index("s") * _NC + lax.axis_index("c")
        base = wid * _RPW
        bufs = (r0, r1, r2)
        gsems = (g0, g1, g2)
        osems = (o0, o1, o2)
        pltpu.sync_copy(idx_hbm.at[pl.ds(base, _RPW)], idx_v)

        def gather_copy(c):
            b = c % 3
            return pltpu.make_async_copy(
                table_hbm.at[idx_v.at[pl.ds(c * _CH, _CH)]], bufs[b], gsems[b]
            )

        def out_copy(c):
            b = c % 3
            return pltpu.make_async_copy(
                bufs[b], out_hbm.at[pl.ds(base + c * _CH, _CH)], osems[b]
            )

        for c in range(min(3, _NCH)):
            gather_copy(c).start()
        for c in range(_NCH):
            gather_copy(c).wait()
            out_copy(c).start()
            if c + 3 < _NCH:
                out_copy(c).wait()
                gather_copy(c + 3).start()
        for c in range(max(_NCH - 3, 0), _NCH):
            out_copy(c).wait()

    return _sc_gather


def _mm_body(g_ref, s_ref, wt_ref, o_ref, st_ref):
    i = pl.program_id(0)
    p = g_ref[...]
    glo = lax.bitcast_convert_type(p << 16, jnp.float32).astype(jnp.bfloat16)
    ghi = lax.bitcast_convert_type(p & jnp.int32(-65536), jnp.float32).astype(jnp.bfloat16)
    s = s_ref[...].astype(jnp.bfloat16)
    out = jnp.dot(glo, wt_ref[:_CSH, :], preferred_element_type=jnp.float32)
    out = out + jnp.dot(ghi, wt_ref[_CSH:_CS, :], preferred_element_type=jnp.float32)
    out = out + jnp.dot(s, wt_ref[_CS:, :], preferred_element_type=jnp.float32)
    o_ref[...] = out.astype(jnp.bfloat16)
    ssum = jnp.sum(out, axis=0, keepdims=True)
    ssq = jnp.sum(out * out, axis=0, keepdims=True)
    acc = jnp.concatenate([ssum, ssq], axis=0)

    @pl.when(i == 0)
    def _():
        st_ref[...] = jnp.zeros_like(st_ref)

    st_ref[0:2, :] = st_ref[0:2, :] + acc


_mm_call = pl.pallas_call(
    _mm_body,
    grid=(_R // _NMM,),
    in_specs=[
        pl.BlockSpec((_NMM, _CSH), lambda i: (i, 0)),
        pl.BlockSpec((_NMM, _CS), lambda i: (i, 0)),
        pl.BlockSpec((_CI, _CO), lambda i: (0, 0)),
    ],
    out_specs=[
        pl.BlockSpec((_NMM, _CO), lambda i: (i, 0)),
        pl.BlockSpec((8, _CO), lambda i: (0, 0)),
    ],
    out_shape=[
        jax.ShapeDtypeStruct((_R, _CO), jnp.bfloat16),
        jax.ShapeDtypeStruct((8, _CO), jnp.float32),
    ],
)


def _norm_body(o_ref, st_ref, g_ref, b_ref, y_ref):
    inv_r = 1.0 / _R
    mean = st_ref[0:1, :] * inv_r
    var = st_ref[1:2, :] * inv_r - mean * mean
    scale = g_ref[0:1, :] / jnp.sqrt(var + 1e-5)
    y = (o_ref[...].astype(jnp.float32) - mean) * scale + b_ref[0:1, :]
    y_ref[...] = jnp.where(y >= 0, y, 0.1 * y)


_norm_call = pl.pallas_call(
    _norm_body,
    grid=(_R // _NMM,),
    in_specs=[
        pl.BlockSpec((_NMM, _CO), lambda i: (i, 0)),
        pl.BlockSpec((8, _CO), lambda i: (0, 0)),
        pl.BlockSpec((1, _CO), lambda i: (0, 0)),
        pl.BlockSpec((1, _CO), lambda i: (0, 0)),
    ],
    out_specs=pl.BlockSpec((_NMM, _CO), lambda i: (i, 0)),
    out_shape=jax.ShapeDtypeStruct((_R, _CO), jnp.float32),
)


def kernel(query_points, support_points, support_features, skip_features, W, bias, gamma, beta):
    qp = jnp.pad(query_points, ((0, 0), (0, 0), (0, 5)))
    qp = qp.reshape(_B * _NBLK, _NBQ, 8)
    spt = jnp.pad(jnp.transpose(support_points, (0, 2, 1)), ((0, 0), (0, 5), (0, 0)))
    idx = _argmin_call(qp, spt).reshape(_R)
    table = _pack_call(support_features.reshape(_B * _M, _CS))
    g = _make_sc_gather()(idx, table)
    skip = skip_features.reshape(_R, _CS)
    out_raw, st = _mm_call(g, skip, W.T.astype(jnp.bfloat16))
    y = _norm_call(out_raw, st, gamma.reshape(1, _CO), beta.reshape(1, _CO))
    return y.reshape(_B, _N, _CO)


# speedup vs baseline: 1.2657x; 1.0039x over previous
import functools

import jax
import jax.numpy as jnp
from jax import lax
from jax.experimental import pallas as pl
from jax.experimental.pallas import tpu as pltpu
from jax.experimental.pallas import tpu_sc as plsc

_B, _N, _M = 4, 8192, 2048
_CS = 512
_CO = 1024
_CI = 2 * _CS
_CSH = _CS // 2
_R = _B * _N
_NBQ = 4096
_NBLK = _N // _NBQ
_NMM = 2048


def _argmin_body(q_ref, st_ref, idx_ref):
    b = pl.program_id(0) // _NBLK
    q = q_ref[0]
    st = st_ref[0]
    st2 = -2.0 * st
    mat = jnp.dot(q, st2, preferred_element_type=jnp.float32)
    qn = jnp.sum(q * q, axis=1, keepdims=True)
    sn = 0.25 * jnp.sum(st2 * st2, axis=0, keepdims=True)
    sq = mat + qn
    sq = sq + sn
    idx = jnp.argmin(sq, axis=1).astype(jnp.int32)[:, None]
    idx_ref[0] = idx + b * _M


_argmin_call = pl.pallas_call(
    _argmin_body,
    grid=(_B * _NBLK,),
    in_specs=[
        pl.BlockSpec((1, _NBQ, 8), lambda i: (i, 0, 0)),
        pl.BlockSpec((1, 8, _M), lambda i: (i // _NBLK, 0, 0)),
    ],
    out_specs=pl.BlockSpec((1, _NBQ, 1), lambda i: (i, 0, 0)),
    out_shape=jax.ShapeDtypeStruct((_B * _NBLK, _NBQ, 1), jnp.int32),
)


def _pack_body(x_ref, o_ref):
    xl = x_ref[:, :_CSH].astype(jnp.bfloat16)
    xh = x_ref[:, _CSH:].astype(jnp.bfloat16)
    bl = lax.bitcast_convert_type(xl, jnp.int16).astype(jnp.int32)
    bh = lax.bitcast_convert_type(xh, jnp.int16).astype(jnp.int32)
    o_ref[...] = (bh << 16) | (bl & 0xFFFF)


_pack_call = pl.pallas_call(
    _pack_body,
    grid=(4,),
    in_specs=[pl.BlockSpec((_B * _M // 4, _CS), lambda i: (i, 0))],
    out_specs=pl.BlockSpec((_B * _M // 4, _CSH), lambda i: (i, 0)),
    out_shape=jax.ShapeDtypeStruct((_B * _M, _CSH), jnp.int32),
)


_NC, _NS = 2, 16
_NW = _NC * _NS
_RPW = _R // _NW
_CH = 128
_NCH = _RPW // _CH

@functools.cache
def _make_sc_gather():
    mesh = plsc.VectorSubcoreMesh(core_axis_name="c", subcore_axis_name="s")

    @functools.partial(
        pl.kernel,
        mesh=mesh,
        out_type=jax.ShapeDtypeStruct((_R, _CSH), jnp.int32),
        scratch_types=[
            pltpu.VMEM((_RPW,), jnp.int32),
            pltpu.VMEM((_CH, _CSH), jnp.int32),
            pltpu.VMEM((_CH, _CSH), jnp.int32),
            pltpu.VMEM((_CH, _CSH), jnp.int32),
            pltpu.SemaphoreType.DMA,
            pltpu.SemaphoreType.DMA,
            pltpu.SemaphoreType.DMA,
            pltpu.SemaphoreType.DMA,
            pltpu.SemaphoreType.DMA,
            pltpu.SemaphoreType.DMA,
        ],
    )
    def _sc_gather(idx_hbm, table_hbm, out_hbm, idx_v, r0, r1, r2, g0, g1, g2, o0, o1, o2):
        wid = lax.axis_index("s") * _NC + lax.axis_index("c")
        base = wid * _RPW
        bufs = (r0, r1, r2)
        gsems = (g0, g1, g2)
        osems = (o0, o1, o2)
        pltpu.sync_copy(idx_hbm.at[pl.ds(base, _RPW)], idx_v)

        def gather_copy(c):
            b = c % 3
            return pltpu.make_async_copy(
                table_hbm.at[idx_v.at[pl.ds(c * _CH, _CH)]], bufs[b], gsems[b]
            )

        def out_copy(c):
            b = c % 3
            return pltpu.make_async_copy(
                bufs[b], out_hbm.at[pl.ds(base + c * _CH, _CH)], osems[b]
            )

        for c in range(min(3, _NCH)):
            gather_copy(c).start()
        for c in range(_NCH):
            gather_copy(c).wait()
            out_copy(c).start()
            if c + 3 < _NCH:
                out_copy(c).wait()
                gather_copy(c + 3).start()
        for c in range(max(_NCH - 3, 0), _NCH):
            out_copy(c).wait()

    return _sc_gather


def _mm_body(g_ref, s_ref, wt_ref, o_ref, st_ref):
    i = pl.program_id(0)
    p = g_ref[...]
    glo = lax.bitcast_convert_type(p << 16, jnp.float32).astype(jnp.bfloat16)
    ghi = lax.bitcast_convert_type(p & jnp.int32(-65536), jnp.float32).astype(jnp.bfloat16)
    s = s_ref[...].astype(jnp.bfloat16)
    out = jnp.dot(glo, wt_ref[:_CSH, :], preferred_element_type=jnp.float32)
    out = out + jnp.dot(ghi, wt_ref[_CSH:_CS, :], preferred_element_type=jnp.float32)
    out = out + jnp.dot(s, wt_ref[_CS:, :], preferred_element_type=jnp.float32)
    o_ref[...] = out.astype(jnp.bfloat16)
    ssum = jnp.sum(out, axis=0, keepdims=True)
    ssq = jnp.sum(out * out, axis=0, keepdims=True)
    acc = jnp.concatenate([ssum, ssq], axis=0)

    @pl.when(i == 0)
    def _():
        st_ref[...] = jnp.zeros_like(st_ref)

    st_ref[0:2, :] = st_ref[0:2, :] + acc


_mm_call = pl.pallas_call(
    _mm_body,
    grid=(_R // _NMM,),
    in_specs=[
        pl.BlockSpec((_NMM, _CSH), lambda i: (i, 0)),
        pl.BlockSpec((_NMM, _CS), lambda i: (i, 0)),
        pl.BlockSpec((_CI, _CO), lambda i: (0, 0)),
    ],
    out_specs=[
        pl.BlockSpec((_NMM, _CO), lambda i: (i, 0)),
        pl.BlockSpec((8, _CO), lambda i: (0, 0)),
    ],
    out_shape=[
        jax.ShapeDtypeStruct((_R, _CO), jnp.bfloat16),
        jax.ShapeDtypeStruct((8, _CO), jnp.float32),
    ],
)


def _norm_body(o_ref, st_ref, g_ref, b_ref, y_ref):
    inv_r = 1.0 / _R
    mean = st_ref[0:1, :] * inv_r
    var = st_ref[1:2, :] * inv_r - mean * mean
    scale = g_ref[0:1, :] / jnp.sqrt(var + 1e-5)
    y = (o_ref[...].astype(jnp.float32) - mean) * scale + b_ref[0:1, :]
    y_ref[...] = jnp.where(y >= 0, y, 0.1 * y)


_norm_call = pl.pallas_call(
    _norm_body,
    grid=(_R // _NMM,),
    in_specs=[
        pl.BlockSpec((_NMM, _CO), lambda i: (i, 0)),
        pl.BlockSpec((8, _CO), lambda i: (0, 0)),
        pl.BlockSpec((1, _CO), lambda i: (0, 0)),
        pl.BlockSpec((1, _CO), lambda i: (0, 0)),
    ],
    out_specs=pl.BlockSpec((_NMM, _CO), lambda i: (i, 0)),
    out_shape=jax.ShapeDtypeStruct((_R, _CO), jnp.float32),
)


def kernel(query_points, support_points, support_features, skip_features, W, bias, gamma, beta):
    qp = jnp.pad(query_points, ((0, 0), (0, 0), (0, 5)))
    qp = qp.reshape(_B * _NBLK, _NBQ, 8)
    spt = jnp.pad(jnp.transpose(support_points, (0, 2, 1)), ((0, 0), (0, 5), (0, 0)))
    idx = _argmin_call(qp, spt).reshape(_R)
    table = _pack_call(support_features.reshape(_B * _M, _CS))
    g = _make_sc_gather()(idx, table)
    skip = skip_features.reshape(_R, _CS)
    out_raw, st = _mm_call(g, skip, W.T.astype(jnp.bfloat16))
    y = _norm_call(out_raw, st, gamma.reshape(1, _CO), beta.reshape(1, _CO))
    return y.reshape(_B, _N, _CO)
